# baseline (speedup 1.0000x reference)
import functools

import jax
import jax.numpy as jnp
from jax import lax
from jax.experimental import pallas as pl
from jax.experimental.pallas import tpu as pltpu
from jax.experimental.pallas import tpu_sc as plsc

V = 1000000
D = 64
R = 16
B = 16384
L = 20

NC = 2
NS = 16
NW = NC * NS
LANES = 16

BL = B * L
IDXW = 128
ROWS_PER_W = BL // NW
IDXROWS_PER_W = ROWS_PER_W // IDXW
CHUNK_IDXROWS = 5
CHUNK_ROWS = CHUNK_IDXROWS * IDXW
NCHUNK = IDXROWS_PER_W // CHUNK_IDXROWS


def _adapter_tc_body(idx_ref, table_ref, wd_ref, bd_ref, wu_ref, bu_ref,
                     out_ref, h_ref, sem):
    cps = [
        pltpu.make_async_copy(
            table_ref.at[pl.ds(idx_ref[l], 1)], h_ref.at[pl.ds(l, 1)], sem)
        for l in range(L)
    ]
    for cp in cps:
        cp.start()
    for cp in cps:
        cp.wait()
    h = h_ref[...]
    mid = jnp.maximum(
        jnp.dot(h, wd_ref[...], preferred_element_type=jnp.float32)
        + bd_ref[...], 0.0)
    out_ref[...] = (
        h + jnp.dot(mid, wu_ref[...], preferred_element_type=jnp.float32)
        + bu_ref[...])


def _adapter_tc(idx0, table, W_down, b_down, W_up, b_up):
    return pl.pallas_call(
        _adapter_tc_body,
        out_shape=jax.ShapeDtypeStruct((L, D), jnp.float32),
        in_specs=[
            pl.BlockSpec(memory_space=pltpu.SMEM),
            pl.BlockSpec(memory_space=pltpu.MemorySpace.HBM),
            pl.BlockSpec(memory_space=pltpu.VMEM),
            pl.BlockSpec(memory_space=pltpu.VMEM),
            pl.BlockSpec(memory_space=pltpu.VMEM),
            pl.BlockSpec(memory_space=pltpu.VMEM),
        ],
        out_specs=pl.BlockSpec(memory_space=pltpu.VMEM),
        scratch_shapes=[
            pltpu.VMEM((L, D), jnp.float32),
            pltpu.SemaphoreType.DMA,
        ],
    )(idx0, table, W_down, b_down.reshape(1, R), W_up, b_up.reshape(1, D))


def _sc_gather_body(idx_hbm, table_hbm, a2_hbm, out_hbm,
                    idx_v, rows_v, a_v, gsem):
    wid = lax.axis_index("s") * NC + lax.axis_index("c")
    pltpu.sync_copy(a2_hbm, a_v)
    idxrow_base = wid * IDXROWS_PER_W
    row_base = wid * ROWS_PER_W

    def chunk_body(ci, carry):
        pltpu.sync_copy(
            idx_hbm.at[pl.ds(idxrow_base + ci * CHUNK_IDXROWS, CHUNK_IDXROWS)],
            idx_v)
        cps = [
            pltpu.async_copy(
                table_hbm.at[idx_v.at[j]],
                rows_v.at[pl.ds(j * IDXW, IDXW)], gsem)
            for j in range(CHUNK_IDXROWS)
        ]
        for cp in cps:
            cp.wait()
        for k in range(D // LANES):
            col = pl.ds(k * LANES, LANES)
            a_regs = [a_v[l, col] for l in range(L)]

            def g_body(g, c, col=col, a_regs=a_regs):
                r0 = g * L
                for l in range(L):
                    rows_v[r0 + l, col] = rows_v[r0 + l, col] + a_regs[l]
                return c

            lax.fori_loop(0, CHUNK_ROWS // L, g_body, 0)
        pltpu.sync_copy(
            rows_v, out_hbm.at[pl.ds(row_base + ci * CHUNK_ROWS, CHUNK_ROWS)])
        return carry

    lax.fori_loop(0, NCHUNK, chunk_body, 0)


_sc_gather = functools.partial(
    pl.kernel,
    mesh=plsc.VectorSubcoreMesh(core_axis_name="c", subcore_axis_name="s"),
    out_type=jax.ShapeDtypeStruct((BL, D), jnp.float32),
    scratch_types=[
        pltpu.VMEM((CHUNK_IDXROWS, IDXW), jnp.int32),
        pltpu.VMEM((CHUNK_ROWS, D), jnp.float32),
        pltpu.VMEM((L, D), jnp.float32),
        pltpu.SemaphoreType.DMA,
    ],
    compiler_params=pltpu.CompilerParams(use_tc_tiling_on_sc=False),
)(_sc_gather_body)


@jax.jit
def kernel(indices, table, W_down, b_down, W_up, b_up):
    a2 = _adapter_tc(indices[0], table, W_down, b_down, W_up, b_up)
    idx2d = indices.reshape(BL // IDXW, IDXW)
    out = _sc_gather(idx2d, table, a2)
    return out.reshape(B, L, D)
